# Initial kernel scaffold; baseline (speedup 1.0000x reference)
import jax
import jax.numpy as jnp
from jax.experimental import pallas as pl


def kernel(x, edge_index, W, b):
    raise NotImplementedError("write your pallas kernel here")


# speedup vs baseline: 26.0115x; 26.0115x over previous
import functools

import jax
import jax.numpy as jnp
from jax import lax
from jax.experimental import pallas as pl
from jax.experimental.pallas import tpu as pltpu
from jax.experimental.pallas import tpu_sc as plsc

N = 10000
D = 128
C = 64
E = 320000
NC = 2
NS = 16
NW = NC * NS
CH = 125
NCHUNK = E // CH // NW
NPAD = 10240
TROWS = NPAD // NS
ZR = 128

_mesh = plsc.VectorSubcoreMesh(
    core_axis_name="c", subcore_axis_name="s", num_cores=NC, num_subcores=NS
)
_sc_params = pltpu.CompilerParams(use_tc_tiling_on_sc=False)


def _fill_rows(ref, nrows, ncol16, val):
    v16 = jnp.full((16,), val, jnp.float32)

    def body(i, carry):
        for j in range(ncol16):
            ref[i, pl.ds(j * 16, 16)] = v16
        return carry

    lax.fori_loop(0, nrows, body, 0)


@functools.partial(
    pl.kernel,
    out_type=jax.ShapeDtypeStruct((NC, NPAD, 16), jnp.float32),
    mesh=_mesh,
    compiler_params=_sc_params,
    scratch_types=[
        pltpu.VMEM((NCHUNK, CH), jnp.int32),
        pltpu.VMEM((CH, 16), jnp.float32),
        pltpu.VMEM((TROWS, 16), jnp.float32),
        pltpu.VMEM_SHARED((NPAD, 16), jnp.float32),
    ],
)
def _deg_kernel(dst_hbm, out_hbm, dst_v, ones_v, zero_v, acc):
    cid = lax.axis_index("c")
    sid = lax.axis_index("s")
    wid = sid * NC + cid
    _fill_rows(ones_v, CH, 1, 1.0)
    _fill_rows(zero_v, TROWS, 1, 0.0)
    pltpu.sync_copy(dst_hbm.at[pl.ds(wid * NCHUNK, NCHUNK)], dst_v)
    pltpu.sync_copy(zero_v, acc.at[pl.ds(sid * TROWS, TROWS)])
    plsc.subcore_barrier()

    def chunk(g, carry):
        pltpu.sync_copy(ones_v, acc.at[dst_v.at[g]], add=True)
        return carry

    lax.fori_loop(0, NCHUNK, chunk, 0)
    plsc.subcore_barrier()
    pltpu.sync_copy(
        acc.at[pl.ds(sid * TROWS, TROWS)], out_hbm.at[cid, pl.ds(sid * TROWS, TROWS)]
    )


@functools.partial(
    pl.kernel,
    out_type=jax.ShapeDtypeStruct((NC, NPAD, C), jnp.float32),
    mesh=_mesh,
    compiler_params=_sc_params,
    scratch_types=[
        pltpu.VMEM((NCHUNK, CH), jnp.int32),
        pltpu.VMEM((NCHUNK, CH), jnp.int32),
        pltpu.VMEM((CH, C), jnp.float32),
        pltpu.VMEM((ZR, C), jnp.float32),
        pltpu.VMEM_SHARED((NPAD, C), jnp.float32),
        pltpu.SemaphoreType.DMA,
    ],
)
def _hop_kernel(z_hbm, src_hbm, dst_hbm, out_hbm, src_v, dst_v, rows_v, zero_v, acc, sem):
    cid = lax.axis_index("c")
    sid = lax.axis_index("s")
    wid = sid * NC + cid
    _fill_rows(zero_v, ZR, C // 16, 0.0)
    pltpu.sync_copy(src_hbm.at[pl.ds(wid * NCHUNK, NCHUNK)], src_v)
    pltpu.sync_copy(dst_hbm.at[pl.ds(wid * NCHUNK, NCHUNK)], dst_v)
    for r in range(TROWS // ZR):
        pltpu.sync_copy(zero_v, acc.at[pl.ds(sid * TROWS + r * ZR, ZR)])
    plsc.subcore_barrier()

    def chunk(g, carry):
        pltpu.async_copy(z_hbm.at[src_v.at[g]], rows_v, sem).wait()
        pltpu.sync_copy(rows_v, acc.at[dst_v.at[g]], add=True)
        return carry

    lax.fori_loop(0, NCHUNK, chunk, 0)
    plsc.subcore_barrier()
    pltpu.sync_copy(
        acc.at[pl.ds(sid * TROWS, TROWS)], out_hbm.at[cid, pl.ds(sid * TROWS, TROWS)]
    )


BLK = 400


def _proj_body(deg_ref, x_ref, w_ref, z_ref):
    deg = deg_ref[0, :, 0:1] + deg_ref[1, :, 0:1] + 1.0
    dinv = lax.rsqrt(deg)
    m = lax.dot_general(
        x_ref[...], w_ref[...], (((1,), (1,)), ((), ())),
        preferred_element_type=jnp.float32,
    )
    z_ref[...] = dinv * m


def _mid_body(deg_ref, z_ref, p_ref, o_ref):
    deg = deg_ref[0, :, 0:1] + deg_ref[1, :, 0:1] + 1.0
    o_ref[...] = (z_ref[...] + p_ref[0] + p_ref[1]) / deg


def _final_body(deg_ref, z2_ref, q_ref, b_ref, o_ref):
    deg = deg_ref[0, :, 0:1] + deg_ref[1, :, 0:1] + 1.0
    dinv = lax.rsqrt(deg)
    t = (z2_ref[...] + q_ref[0] + q_ref[1]) * dinv + b_ref[...]
    mx = jnp.max(t, axis=1, keepdims=True)
    s = t - mx
    lse = jnp.log(jnp.sum(jnp.exp(s), axis=1, keepdims=True))
    o_ref[...] = s - lse


def _deg_spec():
    return pl.BlockSpec((2, BLK, 16), lambda i: (0, i, 0))


_proj = pl.pallas_call(
    _proj_body,
    grid=(N // BLK,),
    in_specs=[
        _deg_spec(),
        pl.BlockSpec((BLK, D), lambda i: (i, 0)),
        pl.BlockSpec((C, D), lambda i: (0, 0)),
    ],
    out_specs=pl.BlockSpec((BLK, C), lambda i: (i, 0)),
    out_shape=jax.ShapeDtypeStruct((N, C), jnp.float32),
)

_mid = pl.pallas_call(
    _mid_body,
    grid=(N // BLK,),
    in_specs=[
        _deg_spec(),
        pl.BlockSpec((BLK, C), lambda i: (i, 0)),
        pl.BlockSpec((2, BLK, C), lambda i: (0, i, 0)),
    ],
    out_specs=pl.BlockSpec((BLK, C), lambda i: (i, 0)),
    out_shape=jax.ShapeDtypeStruct((N, C), jnp.float32),
)

_final = pl.pallas_call(
    _final_body,
    grid=(N // BLK,),
    in_specs=[
        _deg_spec(),
        pl.BlockSpec((BLK, C), lambda i: (i, 0)),
        pl.BlockSpec((2, BLK, C), lambda i: (0, i, 0)),
        pl.BlockSpec((1, C), lambda i: (0, 0)),
    ],
    out_specs=pl.BlockSpec((BLK, C), lambda i: (i, 0)),
    out_shape=jax.ShapeDtypeStruct((N, C), jnp.float32),
)


def kernel(x, edge_index, W, b):
    src = edge_index[0].astype(jnp.int32).reshape(E // CH, CH)
    dst = edge_index[1].astype(jnp.int32).reshape(E // CH, CH)
    degp = _deg_kernel(dst)
    z = _proj(degp, x, W)
    p = _hop_kernel(z, src, dst)
    z2 = _mid(degp, z, p)
    q = _hop_kernel(z2, src, dst)
    return _final(degp, z2, q, b.reshape(1, C))


# speedup vs baseline: 36.5472x; 1.4050x over previous
import functools

import jax
import jax.numpy as jnp
from jax import lax
from jax.experimental import pallas as pl
from jax.experimental.pallas import tpu as pltpu
from jax.experimental.pallas import tpu_sc as plsc

N = 10000
D = 128
C = 64
E = 320000
NC = 2
NS = 16
NW = NC * NS
CH = 125
NCHUNK = E // CH // NW
NPAD = 10240
TROWS = NPAD // NS
ZR = 128

_mesh = plsc.VectorSubcoreMesh(
    core_axis_name="c", subcore_axis_name="s", num_cores=NC, num_subcores=NS
)
_sc_params = pltpu.CompilerParams(use_tc_tiling_on_sc=False)


def _fill_rows(ref, nrows, ncol16, val):
    v16 = jnp.full((16,), val, jnp.float32)

    def body(i, carry):
        for j in range(ncol16):
            ref[i, pl.ds(j * 16, 16)] = v16
        return carry

    lax.fori_loop(0, nrows, body, 0)


@functools.partial(
    pl.kernel,
    out_type=jax.ShapeDtypeStruct((NC, NPAD, 16), jnp.float32),
    mesh=_mesh,
    compiler_params=_sc_params,
    scratch_types=[
        pltpu.VMEM((NCHUNK, CH), jnp.int32),
        pltpu.VMEM((CH, 16), jnp.float32),
        pltpu.VMEM((TROWS, 16), jnp.float32),
        pltpu.VMEM_SHARED((NPAD, 16), jnp.float32),
    ],
)
def _deg_kernel(dst_hbm, out_hbm, dst_v, ones_v, zero_v, acc):
    cid = lax.axis_index("c")
    sid = lax.axis_index("s")
    wid = sid * NC + cid
    _fill_rows(ones_v, CH, 1, 1.0)
    _fill_rows(zero_v, TROWS, 1, 0.0)
    pltpu.sync_copy(dst_hbm.at[pl.ds(wid * NCHUNK, NCHUNK)], dst_v)
    pltpu.sync_copy(zero_v, acc.at[pl.ds(sid * TROWS, TROWS)])
    plsc.subcore_barrier()

    def chunk(g, carry):
        pltpu.sync_copy(ones_v, acc.at[dst_v.at[g]], add=True)
        return carry

    lax.fori_loop(0, NCHUNK, chunk, 0)
    plsc.subcore_barrier()
    pltpu.sync_copy(
        acc.at[pl.ds(sid * TROWS, TROWS)], out_hbm.at[cid, pl.ds(sid * TROWS, TROWS)]
    )


NBUF = 4
NGROUP = NCHUNK // NBUF


@functools.partial(
    pl.kernel,
    out_type=jax.ShapeDtypeStruct((NC, NPAD, C), jnp.float32),
    mesh=_mesh,
    compiler_params=_sc_params,
    scratch_types=[
        pltpu.VMEM((NCHUNK, CH), jnp.int32),
        pltpu.VMEM((NCHUNK, CH), jnp.int32),
        pltpu.VMEM((NBUF, CH, C), jnp.float32),
        pltpu.VMEM((ZR, C), jnp.float32),
        pltpu.VMEM_SHARED((NPAD, C), jnp.float32),
        [pltpu.SemaphoreType.DMA] * NBUF,
        [pltpu.SemaphoreType.DMA] * NBUF,
    ],
)
def _hop_kernel(z_hbm, src_hbm, dst_hbm, out_hbm, src_v, dst_v, rows_v, zero_v, acc,
                gsem, ssem):
    cid = lax.axis_index("c")
    sid = lax.axis_index("s")
    wid = sid * NC + cid
    _fill_rows(zero_v, ZR, C // 16, 0.0)
    pltpu.sync_copy(src_hbm.at[pl.ds(wid * NCHUNK, NCHUNK)], src_v)
    pltpu.sync_copy(dst_hbm.at[pl.ds(wid * NCHUNK, NCHUNK)], dst_v)
    for r in range(TROWS // ZR):
        pltpu.sync_copy(zero_v, acc.at[pl.ds(sid * TROWS + r * ZR, ZR)])
    plsc.subcore_barrier()

    for b in range(NBUF):
        pltpu.async_copy(z_hbm.at[src_v.at[b]], rows_v.at[b], gsem[b])

    def group(k, carry):
        base = k * NBUF
        for b in range(NBUF):
            g = base + b
            pltpu.make_async_copy(z_hbm.at[src_v.at[g]], rows_v.at[b], gsem[b]).wait()
            pltpu.async_copy(rows_v.at[b], acc.at[dst_v.at[g]], ssem[b], add=True)

        @pl.when(k < NGROUP - 1)
        def _():
            for b in range(NBUF):
                g = base + b
                pltpu.make_async_copy(
                    rows_v.at[b], acc.at[dst_v.at[g]], ssem[b]
                ).wait()
                pltpu.async_copy(z_hbm.at[src_v.at[g + NBUF]], rows_v.at[b], gsem[b])

        return carry

    lax.fori_loop(0, NGROUP, group, 0)
    for b in range(NBUF):
        g = NCHUNK - NBUF + b
        pltpu.make_async_copy(rows_v.at[b], acc.at[dst_v.at[g]], ssem[b]).wait()
    plsc.subcore_barrier()
    pltpu.sync_copy(
        acc.at[pl.ds(sid * TROWS, TROWS)], out_hbm.at[cid, pl.ds(sid * TROWS, TROWS)]
    )


BLK = 400


def _proj_body(deg_ref, x_ref, w_ref, z_ref):
    deg = deg_ref[0, :, 0:1] + deg_ref[1, :, 0:1] + 1.0
    dinv = lax.rsqrt(deg)
    m = lax.dot_general(
        x_ref[...], w_ref[...], (((1,), (1,)), ((), ())),
        preferred_element_type=jnp.float32,
    )
    z_ref[...] = dinv * m


def _mid_body(deg_ref, z_ref, p_ref, o_ref):
    deg = deg_ref[0, :, 0:1] + deg_ref[1, :, 0:1] + 1.0
    o_ref[...] = (z_ref[...] + p_ref[0] + p_ref[1]) / deg


def _final_body(deg_ref, z2_ref, q_ref, b_ref, o_ref):
    deg = deg_ref[0, :, 0:1] + deg_ref[1, :, 0:1] + 1.0
    dinv = lax.rsqrt(deg)
    t = (z2_ref[...] + q_ref[0] + q_ref[1]) * dinv + b_ref[...]
    mx = jnp.max(t, axis=1, keepdims=True)
    s = t - mx
    lse = jnp.log(jnp.sum(jnp.exp(s), axis=1, keepdims=True))
    o_ref[...] = s - lse


def _deg_spec():
    return pl.BlockSpec((2, BLK, 16), lambda i: (0, i, 0))


_proj = pl.pallas_call(
    _proj_body,
    grid=(N // BLK,),
    in_specs=[
        _deg_spec(),
        pl.BlockSpec((BLK, D), lambda i: (i, 0)),
        pl.BlockSpec((C, D), lambda i: (0, 0)),
    ],
    out_specs=pl.BlockSpec((BLK, C), lambda i: (i, 0)),
    out_shape=jax.ShapeDtypeStruct((N, C), jnp.float32),
)

_mid = pl.pallas_call(
    _mid_body,
    grid=(N // BLK,),
    in_specs=[
        _deg_spec(),
        pl.BlockSpec((BLK, C), lambda i: (i, 0)),
        pl.BlockSpec((2, BLK, C), lambda i: (0, i, 0)),
    ],
    out_specs=pl.BlockSpec((BLK, C), lambda i: (i, 0)),
    out_shape=jax.ShapeDtypeStruct((N, C), jnp.float32),
)

_final = pl.pallas_call(
    _final_body,
    grid=(N // BLK,),
    in_specs=[
        _deg_spec(),
        pl.BlockSpec((BLK, C), lambda i: (i, 0)),
        pl.BlockSpec((2, BLK, C), lambda i: (0, i, 0)),
        pl.BlockSpec((1, C), lambda i: (0, 0)),
    ],
    out_specs=pl.BlockSpec((BLK, C), lambda i: (i, 0)),
    out_shape=jax.ShapeDtypeStruct((N, C), jnp.float32),
)


def kernel(x, edge_index, W, b):
    src = edge_index[0].astype(jnp.int32).reshape(E // CH, CH)
    dst = edge_index[1].astype(jnp.int32).reshape(E // CH, CH)
    degp = _deg_kernel(dst)
    z = _proj(degp, x, W)
    p = _hop_kernel(z, src, dst)
    z2 = _mid(degp, z, p)
    q = _hop_kernel(z2, src, dst)
    return _final(degp, z2, q, b.reshape(1, C))
